# baseline (speedup 1.0000x reference)
import functools

import jax
import jax.numpy as jnp
from jax import lax
from jax.experimental import pallas as pl
from jax.experimental.pallas import tpu as pltpu
from jax.experimental.pallas import tpu_sc as plsc

N_FULL = 6400000
N_CAND = 2000000
N_SAMP = 500000
N_SEG = 4096
N_PAD = 524288

P1_TILES = 500
P1_T = 4000
P1_MAXT = 16
P2_TILES = 128
P2_T = 4096

_MESH = plsc.VectorSubcoreMesh(core_axis_name="c", subcore_axis_name="s")
_PARAMS = pltpu.CompilerParams(needs_layout_passes=False)
_F32 = jnp.float32
_I32 = jnp.int32
_UNROLL = 8


@functools.partial(
    pl.kernel,
    out_type=(
        jax.ShapeDtypeStruct((2, N_SEG), _F32),
        jax.ShapeDtypeStruct((N_CAND,), _F32),
    ),
    mesh=_MESH,
    compiler_params=_PARAMS,
    scratch_types=[
        pltpu.VMEM((P1_T,), _I32),
        pltpu.VMEM((P1_T,), _I32),
        pltpu.VMEM((P1_T,), _I32),
        pltpu.VMEM((P1_T,), _I32),
        pltpu.VMEM((P1_T,), _I32),
        pltpu.VMEM((P1_T,), _I32),
        pltpu.VMEM((P1_T,), _F32),
        pltpu.VMEM((P1_T,), _F32),
        pltpu.VMEM((P1_T,), _F32),
        pltpu.VMEM((P1_T,), _F32),
        pltpu.VMEM((P1_T,), _F32),
        pltpu.VMEM((P1_T,), _F32),
        pltpu.VMEM((P1_T,), _F32),
        pltpu.VMEM((P1_T,), _F32),
        pltpu.VMEM((P1_T,), _F32),
        pltpu.VMEM((N_SEG,), _F32),
        pltpu.VMEM((N_SEG // 16,), _F32),
        pltpu.VMEM((N_SEG // 16,), _F32),
        pltpu.VMEM_SHARED((16, N_SEG), _F32),
        pltpu.SemaphoreType.DMA,
        pltpu.SemaphoreType.DMA,
        pltpu.SemaphoreType.DMA,
        pltpu.SemaphoreType.DMA,
        pltpu.SemaphoreType.DMA,
        pltpu.SemaphoreType.DMA,
        pltpu.SemaphoreType.DMA,
        pltpu.SemaphoreType.DMA,
        pltpu.SemaphoreType.DMA,
        pltpu.SemaphoreType.DMA,
        pltpu.SemaphoreType.DMA,
        pltpu.SemaphoreType.DMA,
    ],
)
def _phase1(elog, eid, seg, llu, partials, e_all,
            eid_v0, eid_v1, eid_v2, seg_v0, seg_v1, seg_v2,
            llu_v0, llu_v1, llu_v2, lg_v0, lg_v1, lg_v2,
            ev_v0, ev_v1, ev_v2, acc_v, red_a, red_b, accs,
            sem_eid0, sem_eid1, sem_eid2, sem_rest0, sem_rest1, sem_rest2,
            sem_g0, sem_g1, sem_g2, sem_es0, sem_es1, sem_es2):
    c = lax.axis_index("c")
    s = lax.axis_index("s")
    w = s * 2 + c

    eid_v = (eid_v0, eid_v1, eid_v2)
    seg_v = (seg_v0, seg_v1, seg_v2)
    llu_v = (llu_v0, llu_v1, llu_v2)
    lg_v = (lg_v0, lg_v1, lg_v2)
    ev_v = (ev_v0, ev_v1, ev_v2)
    sem_eid = (sem_eid0, sem_eid1, sem_eid2)
    sem_rest = (sem_rest0, sem_rest1, sem_rest2)
    sem_g = (sem_g0, sem_g1, sem_g2)
    sem_es = (sem_es0, sem_es1, sem_es2)

    def zbody(i, carry):
        acc_v[pl.ds(i * 16, 16)] = jnp.zeros((16,), _F32)
        return carry
    lax.fori_loop(0, N_SEG // 16, zbody, 0, unroll=_UNROLL)

    n_extra = P1_TILES - 32 * (P1_TILES // 32)
    nt = jnp.where(w < n_extra, P1_TILES // 32 + 1, P1_TILES // 32)

    def compute(b):
        def body(i, carry):
            k = i * 16
            e = jnp.exp(lg_v[b][pl.ds(k, 16)] + llu_v[b][pl.ds(k, 16)])
            ev_v[b][pl.ds(k, 16)] = e
            plsc.addupdate_scatter(acc_v, [seg_v[b][pl.ds(k, 16)]], e)
            return carry
        lax.fori_loop(0, P1_T // 16, body, 0, unroll=_UNROLL)

    def issue_lin(t, b):
        pltpu.async_copy(eid.at[pl.ds(t * P1_T, P1_T)], eid_v[b], sem_eid[b])
        pltpu.async_copy(seg.at[pl.ds(t * P1_T, P1_T)], seg_v[b], sem_rest[b])
        pltpu.async_copy(llu.at[pl.ds(t * P1_T, P1_T)], llu_v[b], sem_rest[b])

    def wait_eid(t, b):
        pltpu.make_async_copy(
            eid.at[pl.ds(t * P1_T, P1_T)], eid_v[b], sem_eid[b]).wait()

    def wait_rest(t, b):
        pltpu.make_async_copy(
            seg.at[pl.ds(t * P1_T, P1_T)], seg_v[b], sem_rest[b]).wait()
        pltpu.make_async_copy(
            llu.at[pl.ds(t * P1_T, P1_T)], llu_v[b], sem_rest[b]).wait()

    def issue_gather(b):
        return pltpu.async_copy(elog.at[eid_v[b]], lg_v[b], sem_g[b])

    def wait_es(t, b):
        pltpu.make_async_copy(
            ev_v[b], e_all.at[pl.ds(t * P1_T, P1_T)], sem_es[b]).wait()

    issue_lin(w, 0)
    wait_eid(w, 0)
    g_descs = [None, None, None]
    g_descs[0] = issue_gather(0)
    issue_lin(w + 32, 1)

    for j in range(P1_MAXT):
        b = j % 3
        b1 = (j + 1) % 3
        b2 = (j + 2) % 3
        t = w + 32 * j

        def tile_body(j=j, b=b, b1=b1, b2=b2, t=t):
            if j + 1 < P1_MAXT - 1:
                wait_eid(t + 32, b1)
                g_descs[b1] = issue_gather(b1)
            elif j + 1 == P1_MAXT - 1:
                @pl.when(j + 1 < nt)
                def _():
                    wait_eid(t + 32, b1)
                    issue_gather(b1)
            if j + 2 < P1_MAXT - 1:
                issue_lin(t + 64, b2)
            elif j + 2 == P1_MAXT - 1:
                @pl.when(j + 2 < nt)
                def _():
                    issue_lin(t + 64, b2)
            if g_descs[b] is not None and j < P1_MAXT - 1:
                g_descs[b].wait()
            else:
                pltpu.make_async_copy(elog.at[eid_v[b]], lg_v[b],
                                      sem_g[b]).wait()
            wait_rest(t, b)
            if j >= 3:
                wait_es(t - 96, b)
            compute(b)
            pltpu.async_copy(ev_v[b], e_all.at[pl.ds(t * P1_T, P1_T)],
                             sem_es[b])

        if j < P1_MAXT - 1:
            tile_body()
        else:
            @pl.when(j < nt)
            def _():
                tile_body()

    @pl.when(nt == P1_MAXT - 1)
    def _():
        wait_es(w + 32 * 12, 12 % 3)
        wait_es(w + 32 * 13, 13 % 3)
        wait_es(w + 32 * 14, 14 % 3)

    @pl.when(nt == P1_MAXT)
    def _():
        wait_es(w + 32 * 13, 13 % 3)
        wait_es(w + 32 * 14, 14 % 3)
        wait_es(w + 32 * 15, 15 % 3)

    pltpu.sync_copy(acc_v, accs.at[s])
    plsc.subcore_barrier()

    stripe = N_SEG // 16
    pltpu.sync_copy(accs.at[0, pl.ds(s * stripe, stripe)], red_a)
    for r in range(1, 16):
        pltpu.sync_copy(accs.at[r, pl.ds(s * stripe, stripe)], red_b)

        def addb(i, carry):
            k = i * 16
            red_a[pl.ds(k, 16)] = red_a[pl.ds(k, 16)] + red_b[pl.ds(k, 16)]
            return carry
        lax.fori_loop(0, stripe // 16, addb, 0, unroll=_UNROLL)
    pltpu.sync_copy(red_a, partials.at[c, pl.ds(s * stripe, stripe)])


@functools.partial(
    pl.kernel,
    out_type=jax.ShapeDtypeStruct((P2_TILES, P2_T), _F32),
    mesh=_MESH,
    compiler_params=_PARAMS,
    scratch_types=[
        pltpu.VMEM((P2_T,), _I32),
        pltpu.VMEM((P2_T,), _I32),
        pltpu.VMEM((P2_T,), _I32),
        pltpu.VMEM((P2_T,), _I32),
        pltpu.VMEM((P2_T,), _F32),
        pltpu.VMEM((P2_T,), _F32),
        pltpu.VMEM((P2_T,), _F32),
        pltpu.VMEM((P2_T,), _F32),
        pltpu.VMEM((N_SEG,), _F32),
        pltpu.VMEM((N_SEG,), _F32),
        pltpu.SemaphoreType.DMA,
        pltpu.SemaphoreType.DMA,
        pltpu.SemaphoreType.DMA,
        pltpu.SemaphoreType.DMA,
        pltpu.SemaphoreType.DMA,
        pltpu.SemaphoreType.DMA,
        pltpu.SemaphoreType.DMA,
        pltpu.SemaphoreType.DMA,
    ],
)
def _phase2(e_all, partials, sseg, sca, out,
            sseg_v0, sseg_v1, sca_v0, sca_v1, eg_v0, eg_v1, o_v0, o_v1,
            pA, s_v,
            sem_seg0, sem_seg1, sem_ca0, sem_ca1,
            sem_g0, sem_g1, sem_out0, sem_out1):
    c = lax.axis_index("c")
    s = lax.axis_index("s")
    w = s * 2 + c

    sseg_v = (sseg_v0, sseg_v1)
    sca_v = (sca_v0, sca_v1)
    eg_v = (eg_v0, eg_v1)
    o_v = (o_v0, o_v1)
    sem_seg = (sem_seg0, sem_seg1)
    sem_ca = (sem_ca0, sem_ca1)
    sem_g = (sem_g0, sem_g1)
    sem_out = (sem_out0, sem_out1)

    tiles_per_w = P2_TILES // 32
    t0 = w * tiles_per_w

    def issue_lin(t, b):
        pltpu.async_copy(sca.at[t], sca_v[b], sem_ca[b])
        pltpu.async_copy(sseg.at[t], sseg_v[b], sem_seg[b])

    issue_lin(t0, 0)
    issue_lin(t0 + 1, 1)

    pltpu.sync_copy(partials.at[0], pA)
    pltpu.sync_copy(partials.at[1], s_v)

    def rbody(i, carry):
        s_v[pl.ds(i * 16, 16)] = s_v[pl.ds(i * 16, 16)] + pA[pl.ds(i * 16, 16)]
        return carry
    lax.fori_loop(0, N_SEG // 16, rbody, 0, unroll=_UNROLL)

    pltpu.make_async_copy(sca.at[t0], sca_v0, sem_ca0).wait()
    g_descs = [None, None]
    g_descs[0] = pltpu.async_copy(e_all.at[sca_v0], eg_v0, sem_g0)

    out_descs = [None, None]
    for j in range(tiles_per_w):
        b = j % 2
        o = 1 - b
        t = t0 + j
        if j + 1 < tiles_per_w:
            pltpu.make_async_copy(sca.at[t + 1], sca_v[o], sem_ca[o]).wait()
            if out_descs[o] is not None:
                out_descs[o].wait()
            g_descs[o] = pltpu.async_copy(e_all.at[sca_v[o]], eg_v[o],
                                          sem_g[o])
        g_descs[b].wait()
        if j + 2 < tiles_per_w:
            pltpu.async_copy(sca.at[t + 2], sca_v[b], sem_ca[b])
        pltpu.make_async_copy(sseg.at[t], sseg_v[b], sem_seg[b]).wait()

        def vbody(i, carry):
            k = i * 16
            e = eg_v[b][pl.ds(k, 16)]
            sg = plsc.load_gather(s_v, [sseg_v[b][pl.ds(k, 16)]])
            p = e / sg
            o_v[b][pl.ds(k, 16)] = (1.0 - p) + p
            return carry
        lax.fori_loop(0, P2_T // 16, vbody, 0, unroll=_UNROLL)

        out_descs[b] = pltpu.async_copy(o_v[b], out.at[t], sem_out[b])
        if j + 2 < tiles_per_w:
            pltpu.async_copy(sseg.at[t + 2], sseg_v[b], sem_seg[b])

    for d in out_descs:
        if d is not None:
            d.wait()


def kernel(edges_logits, candidate_edges, loglog_u, sampled_edges):
    eid = candidate_edges[:, 1]
    seg = candidate_edges[:, 0]
    partials, e_all = _phase1(edges_logits, eid, seg, loglog_u)

    pad = N_PAD - N_SAMP
    s_seg = jnp.pad(sampled_edges[:, 0], (0, pad)).reshape(P2_TILES, P2_T)
    s_ca = jnp.pad(sampled_edges[:, 5], (0, pad)).reshape(P2_TILES, P2_T)
    out = _phase2(e_all, partials, s_seg, s_ca)
    return out.reshape(-1)[:N_SAMP]


# speedup vs baseline: 1.0391x; 1.0391x over previous
import functools

import jax
import jax.numpy as jnp
from jax import lax
from jax.experimental import pallas as pl
from jax.experimental.pallas import tpu as pltpu
from jax.experimental.pallas import tpu_sc as plsc

N_FULL = 6400000
N_CAND = 2000000
N_SAMP = 500000
N_SEG = 4096
N_PAD = 524288

P1_TILES = 500
P1_T = 4000
P1_MAXT = 16
P2_TILES = 128
P2_T = 4096

_MESH = plsc.VectorSubcoreMesh(core_axis_name="c", subcore_axis_name="s")
_PARAMS = pltpu.CompilerParams(needs_layout_passes=False)
_F32 = jnp.float32
_I32 = jnp.int32
_UNROLL = 8


@functools.partial(
    pl.kernel,
    out_type=(
        jax.ShapeDtypeStruct((2, N_SEG), _F32),
        jax.ShapeDtypeStruct((N_CAND,), _F32),
    ),
    mesh=_MESH,
    compiler_params=_PARAMS,
    scratch_types=[
        pltpu.VMEM((P1_T,), _I32),
        pltpu.VMEM((P1_T,), _I32),
        pltpu.VMEM((P1_T,), _I32),
        pltpu.VMEM((P1_T,), _I32),
        pltpu.VMEM((P1_T,), _I32),
        pltpu.VMEM((P1_T,), _I32),
        pltpu.VMEM((P1_T,), _F32),
        pltpu.VMEM((P1_T,), _F32),
        pltpu.VMEM((P1_T,), _F32),
        pltpu.VMEM((P1_T,), _F32),
        pltpu.VMEM((P1_T,), _F32),
        pltpu.VMEM((P1_T,), _F32),
        pltpu.VMEM((P1_T,), _F32),
        pltpu.VMEM((P1_T,), _F32),
        pltpu.VMEM((P1_T,), _F32),
        pltpu.VMEM((N_SEG,), _F32),
        pltpu.VMEM_SHARED((N_SEG,), _F32),
        pltpu.SemaphoreType.DMA,
        pltpu.SemaphoreType.DMA,
        pltpu.SemaphoreType.DMA,
        pltpu.SemaphoreType.DMA,
        pltpu.SemaphoreType.DMA,
        pltpu.SemaphoreType.DMA,
        pltpu.SemaphoreType.DMA,
        pltpu.SemaphoreType.DMA,
        pltpu.SemaphoreType.DMA,
        pltpu.SemaphoreType.DMA,
        pltpu.SemaphoreType.DMA,
        pltpu.SemaphoreType.DMA,
        pltpu.SemaphoreType.DMA,
        pltpu.SemaphoreType.DMA,
        pltpu.SemaphoreType.DMA,
    ],
)
def _phase1(elog, eid, seg, llu, partials, e_all,
            eid_v0, eid_v1, eid_v2, seg_v0, seg_v1, seg_v2,
            llu_v0, llu_v1, llu_v2, lg_v0, lg_v1, lg_v2,
            ev_v0, ev_v1, ev_v2, z_v, acc_sh,
            sem_eid0, sem_eid1, sem_eid2, sem_rest0, sem_rest1, sem_rest2,
            sem_g0, sem_g1, sem_g2, sem_sc0, sem_sc1, sem_sc2,
            sem_es0, sem_es1, sem_es2):
    c = lax.axis_index("c")
    s = lax.axis_index("s")
    w = s * 2 + c

    eid_v = (eid_v0, eid_v1, eid_v2)
    seg_v = (seg_v0, seg_v1, seg_v2)
    llu_v = (llu_v0, llu_v1, llu_v2)
    lg_v = (lg_v0, lg_v1, lg_v2)
    ev_v = (ev_v0, ev_v1, ev_v2)
    sem_eid = (sem_eid0, sem_eid1, sem_eid2)
    sem_rest = (sem_rest0, sem_rest1, sem_rest2)
    sem_g = (sem_g0, sem_g1, sem_g2)
    sem_sc = (sem_sc0, sem_sc1, sem_sc2)
    sem_es = (sem_es0, sem_es1, sem_es2)

    def zbody(i, carry):
        z_v[pl.ds(i * 16, 16)] = jnp.zeros((16,), _F32)
        return carry
    lax.fori_loop(0, N_SEG // 16, zbody, 0, unroll=_UNROLL)

    @pl.when(s == 0)
    def _():
        pltpu.sync_copy(z_v, acc_sh)

    plsc.subcore_barrier()

    n_extra = P1_TILES - 32 * (P1_TILES // 32)
    nt = jnp.where(w < n_extra, P1_TILES // 32 + 1, P1_TILES // 32)

    def compute(dst, lgb, llb):
        def body(i, carry):
            k = i * 16
            dst[pl.ds(k, 16)] = jnp.exp(lgb[pl.ds(k, 16)] + llb[pl.ds(k, 16)])
            return carry
        lax.fori_loop(0, P1_T // 16, body, 0, unroll=_UNROLL)

    def issue_lin(t, b):
        pltpu.async_copy(eid.at[pl.ds(t * P1_T, P1_T)], eid_v[b], sem_eid[b])
        pltpu.async_copy(seg.at[pl.ds(t * P1_T, P1_T)], seg_v[b], sem_rest[b])
        pltpu.async_copy(llu.at[pl.ds(t * P1_T, P1_T)], llu_v[b], sem_rest[b])

    def wait_eid(t, b):
        pltpu.make_async_copy(
            eid.at[pl.ds(t * P1_T, P1_T)], eid_v[b], sem_eid[b]).wait()

    def wait_rest(t, b):
        pltpu.make_async_copy(
            seg.at[pl.ds(t * P1_T, P1_T)], seg_v[b], sem_rest[b]).wait()
        pltpu.make_async_copy(
            llu.at[pl.ds(t * P1_T, P1_T)], llu_v[b], sem_rest[b]).wait()

    def issue_gather(b):
        return pltpu.async_copy(elog.at[eid_v[b]], lg_v[b], sem_g[b])

    def wait_sc_es(t, b):
        pltpu.make_async_copy(ev_v[b], acc_sh.at[seg_v[b]], sem_sc[b]).wait()
        pltpu.make_async_copy(
            ev_v[b], e_all.at[pl.ds(t * P1_T, P1_T)], sem_es[b]).wait()

    issue_lin(w, 0)
    wait_eid(w, 0)
    g_descs = [None, None, None]
    g_descs[0] = issue_gather(0)
    issue_lin(w + 32, 1)

    for j in range(P1_MAXT):
        b = j % 3
        b1 = (j + 1) % 3
        b2 = (j + 2) % 3
        t = w + 32 * j

        def tile_body(j=j, b=b, b1=b1, b2=b2, t=t):
            if j + 1 < P1_MAXT - 1:
                wait_eid(t + 32, b1)
                g_descs[b1] = issue_gather(b1)
            elif j + 1 == P1_MAXT - 1:
                @pl.when(j + 1 < nt)
                def _():
                    wait_eid(t + 32, b1)
                    issue_gather(b1)
            if j >= 1:
                wait_sc_es(t - 32, b2)
            if j + 2 < P1_MAXT - 1:
                issue_lin(t + 64, b2)
            elif j + 2 == P1_MAXT - 1:
                @pl.when(j + 2 < nt)
                def _():
                    issue_lin(t + 64, b2)
            if g_descs[b] is not None and j < P1_MAXT - 1:
                g_descs[b].wait()
            else:
                pltpu.make_async_copy(elog.at[eid_v[b]], lg_v[b],
                                      sem_g[b]).wait()
            wait_rest(t, b)
            compute(ev_v[b], lg_v[b], llu_v[b])
            pltpu.async_copy(ev_v[b], acc_sh.at[seg_v[b]], sem_sc[b], add=True)
            pltpu.async_copy(ev_v[b], e_all.at[pl.ds(t * P1_T, P1_T)],
                             sem_es[b])

        if j < P1_MAXT - 1:
            tile_body()
        else:
            @pl.when(j < nt)
            def _():
                tile_body()

    @pl.when(nt == P1_MAXT - 1)
    def _():
        wait_sc_es(w + 32 * 14, 14 % 3)

    @pl.when(nt == P1_MAXT)
    def _():
        wait_sc_es(w + 32 * 15, 15 % 3)

    plsc.subcore_barrier()

    @pl.when(s == 0)
    def _():
        pltpu.sync_copy(acc_sh, partials.at[c])


@functools.partial(
    pl.kernel,
    out_type=jax.ShapeDtypeStruct((P2_TILES, P2_T), _F32),
    mesh=_MESH,
    compiler_params=_PARAMS,
    scratch_types=[
        pltpu.VMEM((P2_T,), _I32),
        pltpu.VMEM((P2_T,), _I32),
        pltpu.VMEM((P2_T,), _I32),
        pltpu.VMEM((P2_T,), _I32),
        pltpu.VMEM((P2_T,), _F32),
        pltpu.VMEM((P2_T,), _F32),
        pltpu.VMEM((P2_T,), _F32),
        pltpu.VMEM((P2_T,), _F32),
        pltpu.VMEM((N_SEG,), _F32),
        pltpu.VMEM((N_SEG,), _F32),
        pltpu.SemaphoreType.DMA,
        pltpu.SemaphoreType.DMA,
        pltpu.SemaphoreType.DMA,
        pltpu.SemaphoreType.DMA,
        pltpu.SemaphoreType.DMA,
        pltpu.SemaphoreType.DMA,
        pltpu.SemaphoreType.DMA,
        pltpu.SemaphoreType.DMA,
        pltpu.SemaphoreType.DMA,
        pltpu.SemaphoreType.DMA,
    ],
)
def _phase2(e_all, partials, sseg, sca, out,
            sseg_v0, sseg_v1, sca_v0, sca_v1, eg_v0, eg_v1, o_v0, o_v1,
            pA, s_v,
            sem_seg0, sem_seg1, sem_ca0, sem_ca1,
            sem_g0, sem_g1, sem_out0, sem_out1, sem_gh0, sem_gh1):
    c = lax.axis_index("c")
    s = lax.axis_index("s")
    w = s * 2 + c

    sseg_v = (sseg_v0, sseg_v1)
    sca_v = (sca_v0, sca_v1)
    eg_v = (eg_v0, eg_v1)
    o_v = (o_v0, o_v1)
    sem_seg = (sem_seg0, sem_seg1)
    sem_ca = (sem_ca0, sem_ca1)
    sem_g = (sem_g0, sem_g1)
    sem_out = (sem_out0, sem_out1)
    sem_gh = (sem_gh0, sem_gh1)

    tiles_per_w = P2_TILES // 32
    t0 = w * tiles_per_w

    def issue_lin(t, b):
        pltpu.async_copy(sca.at[t], sca_v[b], sem_ca[b])
        pltpu.async_copy(sseg.at[t], sseg_v[b], sem_seg[b])

    issue_lin(t0, 0)
    issue_lin(t0 + 1, 1)

    pltpu.sync_copy(partials.at[0], pA)
    pltpu.sync_copy(partials.at[1], s_v)

    def rbody(i, carry):
        s_v[pl.ds(i * 16, 16)] = s_v[pl.ds(i * 16, 16)] + pA[pl.ds(i * 16, 16)]
        return carry
    lax.fori_loop(0, N_SEG // 16, rbody, 0, unroll=_UNROLL)

    H = P2_T // 2

    def issue_gather(b):
        d0 = pltpu.async_copy(e_all.at[sca_v[b].at[pl.ds(0, H)]],
                              eg_v[b].at[pl.ds(0, H)], sem_g[b])
        d1 = pltpu.async_copy(e_all.at[sca_v[b].at[pl.ds(H, H)]],
                              eg_v[b].at[pl.ds(H, H)], sem_gh[b])
        return (d0, d1)

    pltpu.make_async_copy(sca.at[t0], sca_v0, sem_ca0).wait()
    g_descs = [None, None]
    g_descs[0] = issue_gather(0)

    out_descs = [None, None]
    for j in range(tiles_per_w):
        b = j % 2
        o = 1 - b
        t = t0 + j
        if j + 1 < tiles_per_w:
            pltpu.make_async_copy(sca.at[t + 1], sca_v[o], sem_ca[o]).wait()
            if out_descs[o] is not None:
                out_descs[o].wait()
            g_descs[o] = issue_gather(o)
        g_descs[b][0].wait()
        g_descs[b][1].wait()
        if j + 2 < tiles_per_w:
            pltpu.async_copy(sca.at[t + 2], sca_v[b], sem_ca[b])
        pltpu.make_async_copy(sseg.at[t], sseg_v[b], sem_seg[b]).wait()

        def vbody(i, carry):
            k = i * 16
            e = eg_v[b][pl.ds(k, 16)]
            sg = plsc.load_gather(s_v, [sseg_v[b][pl.ds(k, 16)]])
            p = e / sg
            o_v[b][pl.ds(k, 16)] = (1.0 - p) + p
            return carry
        lax.fori_loop(0, P2_T // 16, vbody, 0, unroll=_UNROLL)

        out_descs[b] = pltpu.async_copy(o_v[b], out.at[t], sem_out[b])
        if j + 2 < tiles_per_w:
            pltpu.async_copy(sseg.at[t + 2], sseg_v[b], sem_seg[b])

    for d in out_descs:
        if d is not None:
            d.wait()


def kernel(edges_logits, candidate_edges, loglog_u, sampled_edges):
    eid = candidate_edges[:, 1]
    seg = candidate_edges[:, 0]
    pad = N_PAD - N_SAMP
    s_seg = jnp.pad(sampled_edges[:, 0], (0, pad)).reshape(P2_TILES, P2_T)
    s_ca = jnp.pad(sampled_edges[:, 5], (0, pad)).reshape(P2_TILES, P2_T)
    partials, e_all = _phase1(edges_logits, eid, seg, loglog_u)
    out = _phase2(e_all, partials, s_seg, s_ca)
    return out.reshape(-1)[:N_SAMP]


# speedup vs baseline: 1.0415x; 1.0024x over previous
import functools

import jax
import jax.numpy as jnp
from jax import lax
from jax.experimental import pallas as pl
from jax.experimental.pallas import tpu as pltpu
from jax.experimental.pallas import tpu_sc as plsc

N_FULL = 6400000
N_CAND = 2000000
N_SAMP = 500000
N_SEG = 4096
N_PAD = 524288

P1_TILES = 500
P1_T = 4000
P1_MAXT = 16
P2_TILES = 128
P2_T = 4096

_MESH = plsc.VectorSubcoreMesh(core_axis_name="c", subcore_axis_name="s")
_PARAMS = pltpu.CompilerParams(needs_layout_passes=False)
_F32 = jnp.float32
_I32 = jnp.int32
_UNROLL = 8


@functools.partial(
    pl.kernel,
    out_type=(
        jax.ShapeDtypeStruct((2, N_SEG), _F32),
        jax.ShapeDtypeStruct((N_CAND,), _F32),
    ),
    mesh=_MESH,
    compiler_params=_PARAMS,
    scratch_types=[
        pltpu.VMEM((P1_T,), _I32),
        pltpu.VMEM((P1_T,), _I32),
        pltpu.VMEM((P1_T,), _I32),
        pltpu.VMEM((P1_T,), _I32),
        pltpu.VMEM((P1_T,), _I32),
        pltpu.VMEM((P1_T,), _I32),
        pltpu.VMEM((P1_T,), _F32),
        pltpu.VMEM((P1_T,), _F32),
        pltpu.VMEM((P1_T,), _F32),
        pltpu.VMEM((P1_T,), _F32),
        pltpu.VMEM((P1_T,), _F32),
        pltpu.VMEM((P1_T,), _F32),
        pltpu.VMEM((P1_T,), _F32),
        pltpu.VMEM((P1_T,), _F32),
        pltpu.VMEM((P1_T,), _F32),
        pltpu.VMEM((N_SEG,), _F32),
        pltpu.VMEM_SHARED((N_SEG,), _F32),
        pltpu.SemaphoreType.DMA,
        pltpu.SemaphoreType.DMA,
        pltpu.SemaphoreType.DMA,
        pltpu.SemaphoreType.DMA,
        pltpu.SemaphoreType.DMA,
        pltpu.SemaphoreType.DMA,
        pltpu.SemaphoreType.DMA,
        pltpu.SemaphoreType.DMA,
        pltpu.SemaphoreType.DMA,
        pltpu.SemaphoreType.DMA,
        pltpu.SemaphoreType.DMA,
        pltpu.SemaphoreType.DMA,
        pltpu.SemaphoreType.DMA,
        pltpu.SemaphoreType.DMA,
        pltpu.SemaphoreType.DMA,
    ],
)
def _phase1(elog, eid, seg, llu, partials, e_all,
            eid_v0, eid_v1, eid_v2, seg_v0, seg_v1, seg_v2,
            llu_v0, llu_v1, llu_v2, lg_v0, lg_v1, lg_v2,
            ev_v0, ev_v1, ev_v2, z_v, acc_sh,
            sem_eid0, sem_eid1, sem_eid2, sem_rest0, sem_rest1, sem_rest2,
            sem_g0, sem_g1, sem_g2, sem_sc0, sem_sc1, sem_sc2,
            sem_es0, sem_es1, sem_es2):
    c = lax.axis_index("c")
    s = lax.axis_index("s")
    w = s * 2 + c

    eid_v = (eid_v0, eid_v1, eid_v2)
    seg_v = (seg_v0, seg_v1, seg_v2)
    llu_v = (llu_v0, llu_v1, llu_v2)
    lg_v = (lg_v0, lg_v1, lg_v2)
    ev_v = (ev_v0, ev_v1, ev_v2)
    sem_eid = (sem_eid0, sem_eid1, sem_eid2)
    sem_rest = (sem_rest0, sem_rest1, sem_rest2)
    sem_g = (sem_g0, sem_g1, sem_g2)
    sem_sc = (sem_sc0, sem_sc1, sem_sc2)
    sem_es = (sem_es0, sem_es1, sem_es2)

    def zbody(i, carry):
        z_v[pl.ds(i * 16, 16)] = jnp.zeros((16,), _F32)
        return carry
    lax.fori_loop(0, N_SEG // 16, zbody, 0, unroll=_UNROLL)

    @pl.when(s == 0)
    def _():
        pltpu.sync_copy(z_v, acc_sh)

    plsc.subcore_barrier()

    n_extra = P1_TILES - 32 * (P1_TILES // 32)
    nt = jnp.where(w < n_extra, P1_TILES // 32 + 1, P1_TILES // 32)

    def compute(dst, lgb, llb):
        def body(i, carry):
            k = i * 16
            dst[pl.ds(k, 16)] = jnp.exp(lgb[pl.ds(k, 16)] + llb[pl.ds(k, 16)])
            return carry
        lax.fori_loop(0, P1_T // 16, body, 0, unroll=_UNROLL)

    def issue_lin(t, b):
        pltpu.async_copy(eid.at[pl.ds(t * P1_T, P1_T)], eid_v[b], sem_eid[b])
        pltpu.async_copy(seg.at[pl.ds(t * P1_T, P1_T)], seg_v[b], sem_rest[b])
        pltpu.async_copy(llu.at[pl.ds(t * P1_T, P1_T)], llu_v[b], sem_rest[b])

    def wait_eid(t, b):
        pltpu.make_async_copy(
            eid.at[pl.ds(t * P1_T, P1_T)], eid_v[b], sem_eid[b]).wait()

    def wait_rest(t, b):
        pltpu.make_async_copy(
            seg.at[pl.ds(t * P1_T, P1_T)], seg_v[b], sem_rest[b]).wait()
        pltpu.make_async_copy(
            llu.at[pl.ds(t * P1_T, P1_T)], llu_v[b], sem_rest[b]).wait()

    def issue_gather(b):
        return pltpu.async_copy(elog.at[eid_v[b]], lg_v[b], sem_g[b])

    def wait_sc_es(t, b):
        pltpu.make_async_copy(ev_v[b], acc_sh.at[seg_v[b]], sem_sc[b]).wait()
        pltpu.make_async_copy(
            ev_v[b], e_all.at[pl.ds(t * P1_T, P1_T)], sem_es[b]).wait()

    issue_lin(w, 0)
    wait_eid(w, 0)
    g_descs = [None, None, None]
    g_descs[0] = issue_gather(0)
    issue_lin(w + 32, 1)

    for j in range(P1_MAXT):
        b = j % 3
        b1 = (j + 1) % 3
        b2 = (j + 2) % 3
        t = w + 32 * j

        def tile_body(j=j, b=b, b1=b1, b2=b2, t=t):
            if j + 1 < P1_MAXT - 1:
                wait_eid(t + 32, b1)
                g_descs[b1] = issue_gather(b1)
            elif j + 1 == P1_MAXT - 1:
                @pl.when(j + 1 < nt)
                def _():
                    wait_eid(t + 32, b1)
                    issue_gather(b1)
            if j >= 1:
                wait_sc_es(t - 32, b2)
            if j + 2 < P1_MAXT - 1:
                issue_lin(t + 64, b2)
            elif j + 2 == P1_MAXT - 1:
                @pl.when(j + 2 < nt)
                def _():
                    issue_lin(t + 64, b2)
            if g_descs[b] is not None and j < P1_MAXT - 1:
                g_descs[b].wait()
            else:
                pltpu.make_async_copy(elog.at[eid_v[b]], lg_v[b],
                                      sem_g[b]).wait()
            wait_rest(t, b)
            compute(ev_v[b], lg_v[b], llu_v[b])
            pltpu.async_copy(ev_v[b], acc_sh.at[seg_v[b]], sem_sc[b], add=True)
            pltpu.async_copy(ev_v[b], e_all.at[pl.ds(t * P1_T, P1_T)],
                             sem_es[b])

        if j < P1_MAXT - 1:
            tile_body()
        else:
            @pl.when(j < nt)
            def _():
                tile_body()

    @pl.when(nt == P1_MAXT - 1)
    def _():
        wait_sc_es(w + 32 * 14, 14 % 3)

    @pl.when(nt == P1_MAXT)
    def _():
        wait_sc_es(w + 32 * 15, 15 % 3)

    plsc.subcore_barrier()

    @pl.when(s == 0)
    def _():
        pltpu.sync_copy(acc_sh, partials.at[c])


@functools.partial(
    pl.kernel,
    out_type=jax.ShapeDtypeStruct((P2_TILES, P2_T), _F32),
    mesh=_MESH,
    compiler_params=_PARAMS,
    scratch_types=[
        pltpu.VMEM((P2_T,), _I32),
        pltpu.VMEM((P2_T,), _I32),
        pltpu.VMEM((P2_T,), _I32),
        pltpu.VMEM((P2_T,), _I32),
        pltpu.VMEM((P2_T,), _F32),
        pltpu.VMEM((P2_T,), _F32),
        pltpu.VMEM((P2_T,), _F32),
        pltpu.VMEM((P2_T,), _F32),
        pltpu.VMEM((N_SEG,), _F32),
        pltpu.VMEM((N_SEG,), _F32),
        pltpu.SemaphoreType.DMA,
        pltpu.SemaphoreType.DMA,
        pltpu.SemaphoreType.DMA,
        pltpu.SemaphoreType.DMA,
        pltpu.SemaphoreType.DMA,
        pltpu.SemaphoreType.DMA,
        pltpu.SemaphoreType.DMA,
        pltpu.SemaphoreType.DMA,
    ],
)
def _phase2(e_all, partials, sseg, sca, out,
            sseg_v0, sseg_v1, sca_v0, sca_v1, eg_v0, eg_v1, o_v0, o_v1,
            pA, s_v,
            sem_seg0, sem_seg1, sem_ca0, sem_ca1,
            sem_g0, sem_g1, sem_out0, sem_out1):
    c = lax.axis_index("c")
    s = lax.axis_index("s")
    w = s * 2 + c

    sseg_v = (sseg_v0, sseg_v1)
    sca_v = (sca_v0, sca_v1)
    eg_v = (eg_v0, eg_v1)
    o_v = (o_v0, o_v1)
    sem_seg = (sem_seg0, sem_seg1)
    sem_ca = (sem_ca0, sem_ca1)
    sem_g = (sem_g0, sem_g1)
    sem_out = (sem_out0, sem_out1)

    tiles_per_w = P2_TILES // 32
    t0 = w * tiles_per_w

    def issue_lin(t, b):
        pltpu.async_copy(sca.at[t], sca_v[b], sem_ca[b])
        pltpu.async_copy(sseg.at[t], sseg_v[b], sem_seg[b])

    issue_lin(t0, 0)
    issue_lin(t0 + 1, 1)

    pltpu.sync_copy(partials.at[0], pA)
    pltpu.sync_copy(partials.at[1], s_v)

    def rbody(i, carry):
        s_v[pl.ds(i * 16, 16)] = s_v[pl.ds(i * 16, 16)] + pA[pl.ds(i * 16, 16)]
        return carry
    lax.fori_loop(0, N_SEG // 16, rbody, 0, unroll=_UNROLL)

    pltpu.make_async_copy(sca.at[t0], sca_v0, sem_ca0).wait()
    g_descs = [None, None]
    g_descs[0] = pltpu.async_copy(e_all.at[sca_v0], eg_v0, sem_g0)

    out_descs = [None, None]
    for j in range(tiles_per_w):
        b = j % 2
        o = 1 - b
        t = t0 + j
        if j + 1 < tiles_per_w:
            pltpu.make_async_copy(sca.at[t + 1], sca_v[o], sem_ca[o]).wait()
            if out_descs[o] is not None:
                out_descs[o].wait()
            g_descs[o] = pltpu.async_copy(e_all.at[sca_v[o]], eg_v[o],
                                          sem_g[o])
        g_descs[b].wait()
        if j + 2 < tiles_per_w:
            pltpu.async_copy(sca.at[t + 2], sca_v[b], sem_ca[b])
        pltpu.make_async_copy(sseg.at[t], sseg_v[b], sem_seg[b]).wait()

        def vbody(i, carry):
            k = i * 16
            e = eg_v[b][pl.ds(k, 16)]
            sg = plsc.load_gather(s_v, [sseg_v[b][pl.ds(k, 16)]])
            p = e / sg
            o_v[b][pl.ds(k, 16)] = (1.0 - p) + p
            return carry
        lax.fori_loop(0, P2_T // 16, vbody, 0, unroll=_UNROLL)

        out_descs[b] = pltpu.async_copy(o_v[b], out.at[t], sem_out[b])
        if j + 2 < tiles_per_w:
            pltpu.async_copy(sseg.at[t + 2], sseg_v[b], sem_seg[b])

    for d in out_descs:
        if d is not None:
            d.wait()


def kernel(edges_logits, candidate_edges, loglog_u, sampled_edges):
    eid = candidate_edges[:, 1]
    seg = candidate_edges[:, 0]
    partials, e_all = _phase1(edges_logits, eid, seg, loglog_u)

    pad = N_PAD - N_SAMP
    s_seg = jnp.pad(sampled_edges[:, 0], (0, pad)).reshape(P2_TILES, P2_T)
    s_ca = jnp.pad(sampled_edges[:, 5], (0, pad)).reshape(P2_TILES, P2_T)
    out = _phase2(e_all, partials, s_seg, s_ca)
    return out.reshape(-1)[:N_SAMP]
